# baseline (speedup 1.0000x reference)
import functools

import jax
import jax.numpy as jnp
from jax import lax
from jax.experimental import pallas as pl
from jax.experimental.pallas import tpu as pltpu

N = 10000
D_IN = 128
HID = 32
D_OUT = 16
NCHUNK = 8
CHUNK = 1280
LAST = N - (NCHUNK - 1) * CHUNK
SIZES = [CHUNK] * (NCHUNK - 1) + [LAST]

_DNT = (((0,), (1,)), ((), ()))


def _mlp(x_hbm, w0_ref, b0_ref, w1_ref, b1_ref, w2_ref, b2_ref, o_hbm,
         xv, ov, in_sems, out_sem):
    for i in range(NCHUNK):
        pltpu.make_async_copy(
            x_hbm.at[pl.ds(i * CHUNK, SIZES[i]), :],
            xv.at[i, pl.ds(0, SIZES[i]), :],
            in_sems.at[i],
        ).start()
    for i in range(NCHUNK):
        pltpu.make_async_copy(
            x_hbm.at[pl.ds(i * CHUNK, SIZES[i]), :],
            xv.at[i, pl.ds(0, SIZES[i]), :],
            in_sems.at[i],
        ).wait()
        h = jnp.dot(xv[i, pl.ds(0, SIZES[i]), :], w0_ref[...],
                    preferred_element_type=jnp.float32)
        h = jnp.maximum(h + b0_ref[...], 0.0)
        h = jnp.dot(h, w1_ref[...], preferred_element_type=jnp.float32)
        h = jnp.maximum(h + b1_ref[...], 0.0)
        ot = lax.dot_general(w2_ref[...], h, _DNT,
                             preferred_element_type=jnp.float32)
        ov[:, pl.ds(i * CHUNK, SIZES[i])] = ot + b2_ref[...]
    pltpu.make_async_copy(ov, o_hbm, out_sem).start()
    pltpu.make_async_copy(ov, o_hbm, out_sem).wait()


@functools.partial(jax.jit, static_argnames=())
def kernel(x, weight, W0, b0, W1, b1, W2, b2, edge_index, batch):
    del weight, edge_index, batch
    b0r = b0.reshape(1, HID)
    b1r = b1.reshape(1, HID)
    b2c = b2.reshape(D_OUT, 1)
    full = lambda: (0, 0)
    xh = pltpu.with_memory_space_constraint(x, pltpu.MemorySpace.HBM)
    packed = pl.pallas_call(
        _mlp,
        in_specs=[
            pl.BlockSpec(memory_space=pltpu.MemorySpace.HBM),
            pl.BlockSpec((D_IN, HID), full),
            pl.BlockSpec((1, HID), full),
            pl.BlockSpec((HID, HID), full),
            pl.BlockSpec((1, HID), full),
            pl.BlockSpec((HID, D_OUT), full),
            pl.BlockSpec((D_OUT, 1), full),
        ],
        out_specs=pl.BlockSpec(memory_space=pl.ANY),
        out_shape=jax.ShapeDtypeStruct((D_OUT, N), jnp.float32),
        scratch_shapes=[
            pltpu.VMEM((NCHUNK, CHUNK, D_IN), jnp.float32),
            pltpu.VMEM((D_OUT, N), jnp.float32),
            pltpu.SemaphoreType.DMA((NCHUNK,)),
            pltpu.SemaphoreType.DMA,
        ],
    )(xh, W0, b0r, W1, b1r, W2, b2c)
    return packed.T


# speedup vs baseline: 1.0001x; 1.0001x over previous
import functools

import jax
import jax.numpy as jnp
from jax import lax
from jax.experimental import pallas as pl
from jax.experimental.pallas import tpu as pltpu

N = 10000
D_IN = 128
HID = 32
D_OUT = 16
NCHUNK = 8
CHUNK = 1280
LAST = N - (NCHUNK - 1) * CHUNK
SIZES = [CHUNK] * (NCHUNK - 1) + [LAST]

_DNT = (((0,), (1,)), ((), ()))


def _mlp(x_hbm, w0_ref, b0_ref, w1_ref, b1_ref, w2_ref, b2_ref, o_hbm,
         xv0, xv1, xv2, xv3, xv4, xv5, xv6, xv7, ov, in_sems, out_sem):
    xvs = (xv0, xv1, xv2, xv3, xv4, xv5, xv6, xv7)
    for i in range(NCHUNK):
        pltpu.make_async_copy(
            x_hbm.at[pl.ds(i * CHUNK, SIZES[i]), :],
            xvs[i].at[pl.ds(0, SIZES[i]), :],
            in_sems.at[i],
        ).start()
    for i in range(NCHUNK):
        pltpu.make_async_copy(
            x_hbm.at[pl.ds(i * CHUNK, SIZES[i]), :],
            xvs[i].at[pl.ds(0, SIZES[i]), :],
            in_sems.at[i],
        ).wait()
        h = jnp.dot(xvs[i][pl.ds(0, SIZES[i]), :], w0_ref[...],
                    preferred_element_type=jnp.float32)
        h = jnp.maximum(h + b0_ref[...], 0.0)
        h = jnp.dot(h, w1_ref[...], preferred_element_type=jnp.float32)
        h = jnp.maximum(h + b1_ref[...], 0.0)
        ot = lax.dot_general(w2_ref[...], h, _DNT,
                             preferred_element_type=jnp.float32)
        ov[:, pl.ds(i * CHUNK, SIZES[i])] = ot + b2_ref[...]
    pltpu.make_async_copy(ov, o_hbm, out_sem).start()
    pltpu.make_async_copy(ov, o_hbm, out_sem).wait()


@functools.partial(jax.jit, static_argnames=())
def kernel(x, weight, W0, b0, W1, b1, W2, b2, edge_index, batch):
    del weight, edge_index, batch
    b0r = b0.reshape(1, HID)
    b1r = b1.reshape(1, HID)
    b2c = b2.reshape(D_OUT, 1)
    full = lambda: (0, 0)
    xh = pltpu.with_memory_space_constraint(x, pltpu.MemorySpace.HBM)
    packed = pl.pallas_call(
        _mlp,
        in_specs=[
            pl.BlockSpec(memory_space=pltpu.MemorySpace.HBM),
            pl.BlockSpec((D_IN, HID), full),
            pl.BlockSpec((1, HID), full),
            pl.BlockSpec((HID, HID), full),
            pl.BlockSpec((1, HID), full),
            pl.BlockSpec((HID, D_OUT), full),
            pl.BlockSpec((D_OUT, 1), full),
        ],
        out_specs=pl.BlockSpec(memory_space=pl.ANY),
        out_shape=jax.ShapeDtypeStruct((D_OUT, N), jnp.float32),
        scratch_shapes=[
            *[pltpu.VMEM((sz, D_IN), jnp.float32) for sz in SIZES],
            pltpu.VMEM((D_OUT, N), jnp.float32),
            pltpu.SemaphoreType.DMA((NCHUNK,)),
            pltpu.SemaphoreType.DMA,
        ],
    )(xh, W0, b0r, W1, b1r, W2, b2c)
    return packed.T


# speedup vs baseline: 1.2054x; 1.2052x over previous
import functools

import jax
import jax.numpy as jnp
from jax import lax
from jax.experimental import pallas as pl
from jax.experimental.pallas import tpu as pltpu

N = 10000
D_IN = 128
HID = 32
D_OUT = 16

_DNT = (((0,), (1,)), ((), ()))


def _mlp(x_ref, w0_ref, b0_ref, w1_ref, b1_ref, w2_ref, b2_ref, o_hbm,
         ov, out_sem):
    h = jnp.dot(x_ref[...], w0_ref[...], preferred_element_type=jnp.float32)
    h = jnp.maximum(h + b0_ref[...], 0.0)
    h = jnp.dot(h, w1_ref[...], preferred_element_type=jnp.float32)
    h = jnp.maximum(h + b1_ref[...], 0.0)
    ot = lax.dot_general(w2_ref[...], h, _DNT,
                         preferred_element_type=jnp.float32)
    ov[...] = ot + b2_ref[...]
    pltpu.make_async_copy(ov, o_hbm, out_sem).start()
    pltpu.make_async_copy(ov, o_hbm, out_sem).wait()


@functools.partial(jax.jit, static_argnames=())
def kernel(x, weight, W0, b0, W1, b1, W2, b2, edge_index, batch):
    del weight, edge_index, batch
    b0r = b0.reshape(1, HID)
    b1r = b1.reshape(1, HID)
    b2c = b2.reshape(D_OUT, 1)
    vmem = pl.BlockSpec(memory_space=pltpu.MemorySpace.VMEM)
    packed = pl.pallas_call(
        _mlp,
        in_specs=[vmem] * 7,
        out_specs=pl.BlockSpec(memory_space=pl.ANY),
        out_shape=jax.ShapeDtypeStruct((D_OUT, N), jnp.float32),
        scratch_shapes=[
            pltpu.VMEM((D_OUT, N), jnp.float32),
            pltpu.SemaphoreType.DMA,
        ],
    )(x, W0, b0r, W1, b1r, W2, b2c)
    return packed.T
